# baseline (speedup 1.0000x reference)
import jax
import jax.numpy as jnp
from jax import lax
from jax.experimental import pallas as pl
from jax.experimental.pallas import tpu as pltpu
from jax.experimental.pallas import tpu_sc as plsc

N_CLUSTERS = 8192
N_F = 2048
N_T = 512
K_BLK = 512

C_SC = 1024
C_TC = N_CLUSTERS - C_SC
N_BLOCKS = C_TC // K_BLK

SC_NC = 2
SC_NS = 16
NW = SC_NC * SC_NS
CPT = C_SC // NW
CHUNK = 8
NCH = CPT // CHUNK

_LOG_C = (7.0376836292e-2, -1.1514610310e-1, 1.1676998740e-1,
          -1.2420140846e-1, 1.4249322787e-1, -1.6668057665e-1,
          2.0000714765e-1, -2.4999993993e-1, 3.3333331174e-1)
_LN2 = 0.6931471805599453
_SQRT2 = 1.41421356


def _fast_log(v):
    bits = lax.bitcast_convert_type(v, jnp.int32)
    e = lax.shift_right_logical(bits, 23) - 127
    mbits = (bits & 0x007FFFFF) | 0x3F800000
    m = lax.bitcast_convert_type(mbits, jnp.float32)
    cond = m > _SQRT2
    mm = jnp.where(cond, m * 0.5, m)
    ee = e.astype(jnp.float32) + jnp.where(cond, 1.0, 0.0)
    t = mm - 1.0
    z = t * t
    p = jnp.float32(_LOG_C[0])
    for c in _LOG_C[1:]:
        p = p * t + jnp.float32(c)
    return t + (t * z * p - 0.5 * z) + ee * jnp.float32(_LN2)


def _tc_partial_kernel(x_ref, means_ref, vars_ref, part_ref,
                       best_val, best_idx):
    k = pl.program_id(0)

    @pl.when(k == 0)
    def _init():
        best_val[0] = jnp.inf

    x = x_ref[...]
    m = means_ref[...]
    v = vars_ref[...]
    d = m - x
    s = jnp.sum(d * d / v + jnp.log(v), axis=1, keepdims=True)

    bmin = jnp.min(s)
    idx2 = lax.broadcasted_iota(jnp.int32, (K_BLK, 1), 0)
    bidx = jnp.min(jnp.where(s == bmin, idx2, K_BLK))

    @pl.when(bmin < best_val[0])
    def _update():
        best_val[0] = bmin
        best_idx[0] = k * K_BLK + bidx

    @pl.when(k == N_BLOCKS - 1)
    def _emit():
        lane = lax.broadcasted_iota(jnp.int32, (1, 1, 128), 2)
        part_ref[...] = jnp.where(
            lane == 0, best_val[0], best_idx[0].astype(jnp.float32))


def _sc_kernel(mflat, vflat, x_hbm, out_hbm,
               xbuf, bm0, bv0, bm1, bv1, obuf, sem0, sem1, semx):
    nc = lax.axis_index("c")
    ns = lax.axis_index("s")
    wid = ns * SC_NC + nc
    base = C_TC + wid * CPT

    pltpu.async_copy(x_hbm, xbuf, semx).wait()

    bufs = ((bm0, bv0, sem0), (bm1, bv1, sem1))

    def start(ch):
        bm, bv, sem = bufs[ch % 2]
        r0 = (base + ch * CHUNK) * N_F
        c1 = pltpu.async_copy(mflat.at[pl.ds(r0, CHUNK * N_F)], bm, sem)
        c2 = pltpu.async_copy(vflat.at[pl.ds(r0, CHUNK * N_F)], bv, sem)
        return (c1, c2)

    lane = lax.broadcasted_iota(jnp.int32, (16,), 0)

    dnums = lax.GatherDimensionNumbers(
        offset_dims=(), collapsed_slice_dims=(0,), start_index_map=(0,))

    def lane_sum(a):
        for sh in (8, 4, 2, 1):
            perm = lane ^ sh
            a = a + lax.gather(
                a, perm[:, None], dnums, slice_sizes=(1,),
                mode=lax.GatherScatterMode.PROMISE_IN_BOUNDS)
        return a

    pending = {0: start(0)}
    best_s = jnp.full((16,), jnp.inf, jnp.float32)
    best_i = jnp.zeros((16,), jnp.int32)
    for ch in range(NCH):
        if ch + 1 < NCH:
            pending[ch + 1] = start(ch + 1)
        for cp in pending.pop(ch):
            cp.wait()
        bm, bv, _ = bufs[ch % 2]
        for c in range(CHUNK):
            def body(j, acc, _c=c, _bm=bm, _bv=bv):
                off = _c * N_F + j * 16
                m = _bm[pl.ds(off, 16)]
                v = _bv[pl.ds(off, 16)]
                xv = xbuf[pl.ds(j * 16, 16)]
                d = m - xv
                return acc + (d * d / v + _fast_log(v))
            acc = lax.fori_loop(0, N_F // 16, body,
                                jnp.zeros((16,), jnp.float32))
            s = lane_sum(acc)
            gi = jnp.full((16,), base + ch * CHUNK + c, jnp.int32)
            take = s < best_s
            best_s = jnp.where(take, s, best_s)
            best_i = jnp.where(take, gi, best_i)

    obuf[...] = jnp.where(lane == 0, best_s,
                          jnp.where(lane == 1, best_i.astype(jnp.float32),
                                    0.0))
    pltpu.sync_copy(obuf, out_hbm.at[pl.ds(wid * 16, 16)])


_sc_call = pl.kernel(
    _sc_kernel,
    out_type=jax.ShapeDtypeStruct((NW * 16,), jnp.float32),
    mesh=plsc.VectorSubcoreMesh(core_axis_name="c", subcore_axis_name="s"),
    scratch_types=[
        pltpu.VMEM((N_F,), jnp.float32),
        pltpu.VMEM((CHUNK * N_F,), jnp.float32),
        pltpu.VMEM((CHUNK * N_F,), jnp.float32),
        pltpu.VMEM((CHUNK * N_F,), jnp.float32),
        pltpu.VMEM((CHUNK * N_F,), jnp.float32),
        pltpu.VMEM((16,), jnp.float32),
        pltpu.SemaphoreType.DMA,
        pltpu.SemaphoreType.DMA,
        pltpu.SemaphoreType.DMA,
    ],
)


def _merge_kernel(tc_ref, sc_ref, y_means_ref, out_ref, sem):
    v = tc_ref[0, 0, 0]
    i = tc_ref[0, 0, 1]
    for w in range(NW):
        vw = sc_ref[w, 0]
        iw = sc_ref[w, 1]
        pred = vw < v
        v = jnp.where(pred, vw, v)
        i = jnp.where(pred, iw, i)
    idx = i.astype(jnp.int32)
    cp = pltpu.make_async_copy(
        y_means_ref.at[pl.ds(idx, 1), :], out_ref, sem)
    cp.start()
    cp.wait()


def kernel(t, x, means, vars_, y_means, y_vars):
    tc_part = pl.pallas_call(
        _tc_partial_kernel,
        grid=(N_BLOCKS,),
        in_specs=[
            pl.BlockSpec((1, N_F), lambda k: (0, 0)),
            pl.BlockSpec((K_BLK, N_F), lambda k: (k, 0)),
            pl.BlockSpec((K_BLK, N_F), lambda k: (k, 0)),
        ],
        out_specs=pl.BlockSpec((1, 1, 128), lambda k: (0, 0, 0)),
        out_shape=jax.ShapeDtypeStruct((1, 1, 128), jnp.float32),
        scratch_shapes=[
            pltpu.SMEM((1,), jnp.float32),
            pltpu.SMEM((1,), jnp.int32),
        ],
        compiler_params=pltpu.CompilerParams(
            dimension_semantics=("arbitrary",),
        ),
    )(x.reshape(1, N_F), means, vars_)

    sc_part = _sc_call(means.reshape(-1), vars_.reshape(-1), x)

    out = pl.pallas_call(
        _merge_kernel,
        in_specs=[
            pl.BlockSpec(memory_space=pltpu.SMEM),
            pl.BlockSpec(memory_space=pltpu.SMEM),
            pl.BlockSpec(memory_space=pl.ANY),
        ],
        out_specs=pl.BlockSpec(memory_space=pltpu.VMEM),
        out_shape=jax.ShapeDtypeStruct((1, N_T), jnp.float32),
        scratch_shapes=[pltpu.SemaphoreType.DMA],
    )(tc_part, sc_part.reshape(NW, 16), y_means)
    return out.reshape(N_T)


# speedup vs baseline: 2.2973x; 2.2973x over previous
import jax
import jax.numpy as jnp
from jax import lax
from jax.experimental import pallas as pl
from jax.experimental.pallas import tpu as pltpu
from jax.experimental.pallas import tpu_sc as plsc

N_CLUSTERS = 8192
N_F = 2048
N_T = 512
K_BLK = 512

C_SC = 1024
C_TC = N_CLUSTERS - C_SC
N_BLOCKS = C_TC // K_BLK

SC_NC = 2
SC_NS = 16
NW = SC_NC * SC_NS
CPT = C_SC // NW
CHUNK = 8
NCH = CPT // CHUNK
UNROLL = 4

_LOG_C = (7.0376836292e-2, -1.1514610310e-1, 1.1676998740e-1,
          -1.2420140846e-1, 1.4249322787e-1, -1.6668057665e-1,
          2.0000714765e-1, -2.4999993993e-1, 3.3333331174e-1)
_LN2 = 0.6931471805599453
_SQRT2 = 1.41421356


def _fast_log(v):
    bits = lax.bitcast_convert_type(v, jnp.int32)
    e = lax.shift_right_logical(bits, 23) - 127
    mbits = (bits & 0x007FFFFF) | 0x3F800000
    m = lax.bitcast_convert_type(mbits, jnp.float32)
    cond = m > _SQRT2
    mm = jnp.where(cond, m * 0.5, m)
    ee = e.astype(jnp.float32) + jnp.where(cond, 1.0, 0.0)
    t = mm - 1.0
    z = t * t
    p = jnp.float32(_LOG_C[0])
    for c in _LOG_C[1:]:
        p = p * t + jnp.float32(c)
    return t + (t * z * p - 0.5 * z) + ee * jnp.float32(_LN2)


def _tc_partial_kernel(x_ref, means_ref, vars_ref, part_ref,
                       best_val, best_idx):
    k = pl.program_id(0)

    @pl.when(k == 0)
    def _init():
        best_val[0] = jnp.inf

    x = x_ref[...]
    m = means_ref[...]
    v = vars_ref[...]
    d = m - x
    s = jnp.sum(d * d / v + jnp.log(v), axis=1, keepdims=True)

    bmin = jnp.min(s)
    idx2 = lax.broadcasted_iota(jnp.int32, (K_BLK, 1), 0)
    bidx = jnp.min(jnp.where(s == bmin, idx2, K_BLK))

    @pl.when(bmin < best_val[0])
    def _update():
        best_val[0] = bmin
        best_idx[0] = k * K_BLK + bidx

    @pl.when(k == N_BLOCKS - 1)
    def _emit():
        lane = lax.broadcasted_iota(jnp.int32, (1, 1, 128), 2)
        part_ref[...] = jnp.where(
            lane == 0, best_val[0], best_idx[0].astype(jnp.float32))


def _sc_kernel(mflat, vflat, x_hbm, out_hbm,
               xbuf, bm0, bv0, bm1, bv1, obuf, sem0, sem1, semx):
    nc = lax.axis_index("c")
    ns = lax.axis_index("s")
    wid = ns * SC_NC + nc
    base = C_TC + wid * CPT

    pltpu.async_copy(x_hbm, xbuf, semx).wait()

    bufs = ((bm0, bv0, sem0), (bm1, bv1, sem1))

    def start(ch):
        bm, bv, sem = bufs[ch % 2]
        r0 = base + ch * CHUNK
        c1 = pltpu.async_copy(mflat.at[pl.ds(r0, CHUNK)], bm, sem)
        c2 = pltpu.async_copy(vflat.at[pl.ds(r0, CHUNK)], bv, sem)
        return (c1, c2)

    lane = lax.broadcasted_iota(jnp.int32, (16,), 0)

    dnums = lax.GatherDimensionNumbers(
        offset_dims=(), collapsed_slice_dims=(0,), start_index_map=(0,))

    def lane_sum(a):
        for sh in (8, 4, 2, 1):
            perm = lane ^ sh
            a = a + lax.gather(
                a, perm[:, None], dnums, slice_sizes=(1,),
                mode=lax.GatherScatterMode.PROMISE_IN_BOUNDS)
        return a

    pending = {0: start(0)}
    best_s = jnp.full((16,), jnp.inf, jnp.float32)
    best_i = jnp.zeros((16,), jnp.int32)
    for ch in range(NCH):
        if ch + 1 < NCH:
            pending[ch + 1] = start(ch + 1)
        for cp in pending.pop(ch):
            cp.wait()
        bm, bv, _ = bufs[ch % 2]
        for c in range(CHUNK):
            def body(j, accs, _c=c, _bm=bm, _bv=bv):
                new = []
                for u in range(UNROLL):
                    off = (j * UNROLL + u) * 16
                    m = _bm[_c, pl.ds(off, 16)]
                    v = _bv[_c, pl.ds(off, 16)]
                    xv = xbuf[pl.ds(off, 16)]
                    d = m - xv
                    new.append(accs[u] + (d * d / v + _fast_log(v)))
                return tuple(new)
            accs = lax.fori_loop(
                0, N_F // (16 * UNROLL), body,
                tuple(jnp.zeros((16,), jnp.float32) for _ in range(UNROLL)))
            acc = accs[0]
            for u in range(1, UNROLL):
                acc = acc + accs[u]
            s = lane_sum(acc)
            gi = jnp.full((16,), base + ch * CHUNK + c, jnp.int32)
            take = s < best_s
            best_s = jnp.where(take, s, best_s)
            best_i = jnp.where(take, gi, best_i)

    obuf[...] = jnp.where(lane == 0, best_s,
                          jnp.where(lane == 1, best_i.astype(jnp.float32),
                                    0.0))
    pltpu.sync_copy(obuf, out_hbm.at[pl.ds(wid * 16, 16)])


_sc_call = pl.kernel(
    _sc_kernel,
    out_type=jax.ShapeDtypeStruct((NW * 16,), jnp.float32),
    mesh=plsc.VectorSubcoreMesh(core_axis_name="c", subcore_axis_name="s"),
    scratch_types=[
        pltpu.VMEM((N_F,), jnp.float32),
        pltpu.VMEM((CHUNK, N_F), jnp.float32),
        pltpu.VMEM((CHUNK, N_F), jnp.float32),
        pltpu.VMEM((CHUNK, N_F), jnp.float32),
        pltpu.VMEM((CHUNK, N_F), jnp.float32),
        pltpu.VMEM((16,), jnp.float32),
        pltpu.SemaphoreType.DMA,
        pltpu.SemaphoreType.DMA,
        pltpu.SemaphoreType.DMA,
    ],
)


def _merge_kernel(tc_ref, sc_ref, y_means_ref, out_ref, sem):
    v = tc_ref[0, 0, 0]
    i = tc_ref[0, 0, 1]
    for w in range(NW):
        vw = sc_ref[w, 0]
        iw = sc_ref[w, 1]
        pred = vw < v
        v = jnp.where(pred, vw, v)
        i = jnp.where(pred, iw, i)
    idx = i.astype(jnp.int32)
    cp = pltpu.make_async_copy(
        y_means_ref.at[pl.ds(idx, 1), :], out_ref, sem)
    cp.start()
    cp.wait()


def kernel(t, x, means, vars_, y_means, y_vars):
    tc_part = pl.pallas_call(
        _tc_partial_kernel,
        grid=(N_BLOCKS,),
        in_specs=[
            pl.BlockSpec((1, N_F), lambda k: (0, 0)),
            pl.BlockSpec((K_BLK, N_F), lambda k: (k, 0)),
            pl.BlockSpec((K_BLK, N_F), lambda k: (k, 0)),
        ],
        out_specs=pl.BlockSpec((1, 1, 128), lambda k: (0, 0, 0)),
        out_shape=jax.ShapeDtypeStruct((1, 1, 128), jnp.float32),
        scratch_shapes=[
            pltpu.SMEM((1,), jnp.float32),
            pltpu.SMEM((1,), jnp.int32),
        ],
        compiler_params=pltpu.CompilerParams(
            dimension_semantics=("arbitrary",),
        ),
    )(x.reshape(1, N_F), means, vars_)

    sc_part = _sc_call(means, vars_, x)

    out = pl.pallas_call(
        _merge_kernel,
        in_specs=[
            pl.BlockSpec(memory_space=pltpu.SMEM),
            pl.BlockSpec(memory_space=pltpu.SMEM),
            pl.BlockSpec(memory_space=pl.ANY),
        ],
        out_specs=pl.BlockSpec(memory_space=pltpu.VMEM),
        out_shape=jax.ShapeDtypeStruct((1, N_T), jnp.float32),
        scratch_shapes=[pltpu.SemaphoreType.DMA],
    )(tc_part, sc_part.reshape(NW, 16), y_means)
    return out.reshape(N_T)


# speedup vs baseline: 4.0311x; 1.7547x over previous
import jax
import jax.numpy as jnp
from jax import lax
from jax.experimental import pallas as pl
from jax.experimental.pallas import tpu as pltpu

N_CLUSTERS = 8192
N_F = 2048
N_T = 512
K_BLK = 512
N_BLOCKS = N_CLUSTERS // K_BLK


def _em_kernel(x_ref, means_ref, vars_ref, y_means_ref, out_ref,
               best_val, best_idx, sem):
    k = pl.program_id(0)

    @pl.when(k == 0)
    def _init():
        best_val[0] = jnp.inf

    x = x_ref[...]
    m = means_ref[...]
    v = vars_ref[...]
    d = m - x
    s = jnp.sum(d * d / v + jnp.log(v), axis=1, keepdims=True)

    bmin = jnp.min(s)
    idx2 = lax.broadcasted_iota(jnp.int32, (K_BLK, 1), 0)
    bidx = jnp.min(jnp.where(s == bmin, idx2, K_BLK))

    @pl.when(bmin < best_val[0])
    def _update():
        best_val[0] = bmin
        best_idx[0] = k * K_BLK + bidx

    @pl.when(k == N_BLOCKS - 1)
    def _gather():
        i = best_idx[0]
        cp = pltpu.make_async_copy(
            y_means_ref.at[pl.ds(i, 1), :], out_ref, sem)
        cp.start()
        cp.wait()


def kernel(t, x, means, vars_, y_means, y_vars):
    out = pl.pallas_call(
        _em_kernel,
        grid=(N_BLOCKS,),
        in_specs=[
            pl.BlockSpec((1, N_F), lambda k: (0, 0)),
            pl.BlockSpec((K_BLK, N_F), lambda k: (k, 0)),
            pl.BlockSpec((K_BLK, N_F), lambda k: (k, 0)),
            pl.BlockSpec(memory_space=pl.ANY),
        ],
        out_specs=pl.BlockSpec((1, N_T), lambda k: (0, 0)),
        out_shape=jax.ShapeDtypeStruct((1, N_T), jnp.float32),
        scratch_shapes=[
            pltpu.SMEM((1,), jnp.float32),
            pltpu.SMEM((1,), jnp.int32),
            pltpu.SemaphoreType.DMA,
        ],
        compiler_params=pltpu.CompilerParams(
            dimension_semantics=("arbitrary",),
        ),
    )(x.reshape(1, N_F), means, vars_, y_means)
    return out.reshape(N_T)
